# baseline (speedup 1.0000x reference)
import functools

import jax
import jax.numpy as jnp
from jax import lax
from jax.experimental import pallas as pl
from jax.experimental.pallas import tpu as pltpu
from jax.experimental.pallas import tpu_sc as plsc

LANES = 16
SLAB = 128
SLABS_PER_CHUNK = 8
CHUNK = SLAB * SLABS_PER_CHUNK


def _build_lookup(s_dim: int, b_dim: int, embed: int, num_workers: int):
    n_slabs = s_dim * (b_dim // SLAB)
    slabs_per_worker = n_slabs // num_workers
    n_chunks = slabs_per_worker // SLABS_PER_CHUNK
    tb_dim = b_dim // SLAB
    te_dim = embed // 8

    mesh = plsc.VectorSubcoreMesh(core_axis_name="c", subcore_axis_name="s")

    @functools.partial(
        pl.kernel,
        mesh=mesh,
        compiler_params=pltpu.CompilerParams(
            use_tc_tiling_on_sc=False, needs_layout_passes=False),
        out_type=jax.ShapeDtypeStruct((s_dim, te_dim, tb_dim, 8, SLAB),
                                      jnp.float32),
        scratch_types=[
            pltpu.VMEM((SLABS_PER_CHUNK, SLAB), jnp.int32),
            pltpu.VMEM((CHUNK, embed), jnp.float32),
            pltpu.VMEM((SLABS_PER_CHUNK, te_dim, 8, SLAB), jnp.float32),
            pltpu.SemaphoreType.DMA,
            pltpu.SemaphoreType.DMA,
        ],
    )
    def lookup(table_hbm, idx_hbm, out_hbm, idx_v, raw_v, tout_v, gsem, wsem):
        n_cores = lax.axis_size("c")
        wid = lax.axis_index("s") * n_cores + lax.axis_index("c")
        slab_base = wid * slabs_per_worker
        iot = lax.iota(jnp.int32, LANES)
        zeros16 = jnp.zeros((LANES,), jnp.float32)

        def do_chunk(g, carry):
            chunk_slab = slab_base + g * SLABS_PER_CHUNK
            pltpu.sync_copy(idx_hbm.at[pl.ds(chunk_slab, SLABS_PER_CHUNK)],
                            idx_v)
            gathers = [
                pltpu.async_copy(
                    table_hbm.at[idx_v.at[j]],
                    raw_v.at[pl.ds(j * SLAB, SLAB)],
                    gsem,
                )
                for j in range(SLABS_PER_CHUNK)
            ]
            for cp in gathers:
                cp.wait()

            def transpose_slab(j, c):
                for bg in range(SLAB // LANES):
                    v = idx_v[j, pl.ds(bg * LANES, LANES)]
                    m = v != 0
                    rows = iot + (j * SLAB + bg * LANES)
                    for e in range(embed):
                        col = jnp.full((LANES,), e, jnp.int32)
                        vals = plsc.load_gather(raw_v, [rows, col])
                        tout_v[j, e // 8, e % 8,
                               pl.ds(bg * LANES, LANES)] = jnp.where(
                                   m, vals, zeros16)
                return c

            lax.fori_loop(0, SLABS_PER_CHUNK, transpose_slab, 0)

            writes = []
            for j in range(SLABS_PER_CHUNK):
                sj = (chunk_slab + j) // tb_dim
                tbj = (chunk_slab + j) % tb_dim
                for te in range(te_dim):
                    writes.append(pltpu.async_copy(
                        tout_v.at[j, te], out_hbm.at[sj, te, tbj], wsem))
            for cp in writes:
                cp.wait()
            return carry

        lax.fori_loop(0, n_chunks, do_chunk, 0)

    return lookup


def kernel(instruction, table):
    b, s = instruction.shape
    vocab, embed = table.shape
    idx = instruction.astype(jnp.int32).T.reshape(s * (b // SLAB), SLAB)
    info = plsc.get_sparse_core_info()
    num_workers = info.num_cores * info.num_subcores
    out5 = _build_lookup(s, b, embed, num_workers)(table, idx)
    return out5.transpose(2, 4, 0, 1, 3).reshape(b, s, embed)


# speedup vs baseline: 1.4781x; 1.4781x over previous
import functools

import jax
import jax.numpy as jnp
from jax import lax
from jax.experimental import pallas as pl
from jax.experimental.pallas import tpu as pltpu
from jax.experimental.pallas import tpu_sc as plsc

LANES = 16
SLAB = 128
SLABS_PER_CHUNK = 8
CHUNK = SLAB * SLABS_PER_CHUNK


def _build_lookup(s_dim: int, b_dim: int, embed: int, num_workers: int):
    tb_dim = b_dim // SLAB
    te_dim = embed // 8
    n_slabs = s_dim * tb_dim
    slabs_per_worker = n_slabs // num_workers
    n_chunks = slabs_per_worker // SLABS_PER_CHUNK
    tout_len = te_dim * SLABS_PER_CHUNK * 8 * SLAB
    out_len = b_dim * s_dim * embed

    mesh = plsc.VectorSubcoreMesh(core_axis_name="c", subcore_axis_name="s")

    @functools.partial(
        pl.kernel,
        mesh=mesh,
        compiler_params=pltpu.CompilerParams(
            use_tc_tiling_on_sc=False, needs_layout_passes=False),
        out_type=jax.ShapeDtypeStruct((out_len,), jnp.float32),
        scratch_types=[
            pltpu.VMEM((SLABS_PER_CHUNK, SLAB), jnp.int32),
            pltpu.VMEM((CHUNK, embed), jnp.float32),
            pltpu.VMEM((tout_len,), jnp.float32),
            pltpu.SemaphoreType.DMA,
            pltpu.SemaphoreType.DMA,
        ],
    )
    def lookup(table_hbm, idx_hbm, out_hbm, idx_v, raw_v, tout_v, gsem, wsem):
        n_cores = lax.axis_size("c")
        wid = lax.axis_index("s") * n_cores + lax.axis_index("c")
        slab_base = wid * slabs_per_worker
        iot = lax.iota(jnp.int32, LANES)
        zeros16 = jnp.zeros((LANES,), jnp.float32)

        def do_chunk(g, carry):
            chunk_slab = slab_base + g * SLABS_PER_CHUNK
            pltpu.sync_copy(idx_hbm.at[pl.ds(chunk_slab, SLABS_PER_CHUNK)],
                            idx_v)
            gathers = [
                pltpu.async_copy(
                    table_hbm.at[idx_v.at[j]],
                    raw_v.at[pl.ds(j * SLAB, SLAB)],
                    gsem,
                )
                for j in range(SLABS_PER_CHUNK)
            ]
            for cp in gathers:
                cp.wait()

            def transpose_slab(j, c):
                for bg in range(SLAB // LANES):
                    m = idx_v[j, pl.ds(bg * LANES, LANES)] != 0
                    rowbase = j * SLAB + bg * LANES
                    posbase = j * (8 * SLAB) + bg * LANES
                    rows = iot + rowbase
                    for e0 in (0, LANES):
                        for k in range(LANES):
                            e_vec = ((iot + k) & (LANES - 1)) + e0
                            pos_static = (((e_vec >> 3) << 13)
                                          + ((e_vec & 7) << 7) + iot)
                            vals = plsc.load_gather(raw_v, [rows, e_vec])
                            plsc.store_scatter(
                                tout_v, [pos_static + posbase],
                                jnp.where(m, vals, zeros16))
                return c

            lax.fori_loop(0, SLABS_PER_CHUNK, transpose_slab, 0)

            s_idx = chunk_slab // tb_dim
            qoff = (chunk_slab % tb_dim) * (8 * SLAB)
            writes = [
                pltpu.async_copy(
                    tout_v.at[pl.ds(te * (SLABS_PER_CHUNK * 8 * SLAB),
                                    SLABS_PER_CHUNK * 8 * SLAB)],
                    out_hbm.at[pl.ds(
                        s_idx * (te_dim * tb_dim * 8 * SLAB)
                        + te * (tb_dim * 8 * SLAB) + qoff,
                        SLABS_PER_CHUNK * 8 * SLAB)],
                    wsem,
                )
                for te in range(te_dim)
            ]
            for cp in writes:
                cp.wait()
            return carry

        lax.fori_loop(0, n_chunks, do_chunk, 0)

    return lookup


def kernel(instruction, table):
    b, s = instruction.shape
    vocab, embed = table.shape
    idx = instruction.astype(jnp.int32).T.reshape(s * (b // SLAB), SLAB)
    info = plsc.get_sparse_core_info()
    num_workers = info.num_cores * info.num_subcores
    flat = _build_lookup(s, b, embed, num_workers)(table, idx)
    out5 = flat.reshape(s, embed // 8, b // SLAB, 8, SLAB)
    return out5.transpose(2, 4, 0, 1, 3).reshape(b, s, embed)


# speedup vs baseline: 1.4785x; 1.0003x over previous
import functools

import jax
import jax.numpy as jnp
from jax import lax
from jax.experimental import pallas as pl
from jax.experimental.pallas import tpu as pltpu
from jax.experimental.pallas import tpu_sc as plsc

LANES = 16
SLAB = 128
SLABS_PER_CHUNK = 8
CHUNK = SLAB * SLABS_PER_CHUNK


def _build_lookup(s_dim: int, b_dim: int, embed: int, num_workers: int):
    tb_dim = b_dim // SLAB
    te_dim = embed // 8
    n_slabs = s_dim * tb_dim
    slabs_per_worker = n_slabs // num_workers
    n_chunks = slabs_per_worker // SLABS_PER_CHUNK
    tout_len = te_dim * SLABS_PER_CHUNK * 8 * SLAB
    out_len = b_dim * s_dim * embed

    mesh = plsc.VectorSubcoreMesh(core_axis_name="c", subcore_axis_name="s")

    @functools.partial(
        pl.kernel,
        mesh=mesh,
        compiler_params=pltpu.CompilerParams(
            use_tc_tiling_on_sc=False, needs_layout_passes=False),
        out_type=jax.ShapeDtypeStruct((out_len,), jnp.float32),
        scratch_types=[
            pltpu.VMEM((SLABS_PER_CHUNK, SLAB), jnp.int32),
            pltpu.VMEM((CHUNK, embed), jnp.float32),
            pltpu.VMEM((tout_len,), jnp.float32),
            pltpu.SemaphoreType.DMA,
            pltpu.SemaphoreType.DMA,
        ],
    )
    def lookup(table_hbm, idx_hbm, out_hbm, idx_v, raw_v, tout_v, gsem, wsem):
        n_cores = lax.axis_size("c")
        wid = lax.axis_index("s") * n_cores + lax.axis_index("c")
        slab_base = wid * slabs_per_worker
        iot = lax.iota(jnp.int32, LANES)
        zeros16 = jnp.zeros((LANES,), jnp.float32)

        def do_chunk(g, carry):
            chunk_slab = slab_base + g * SLABS_PER_CHUNK
            pltpu.sync_copy(idx_hbm.at[pl.ds(chunk_slab, SLABS_PER_CHUNK)],
                            idx_v)
            gathers = [
                pltpu.async_copy(
                    table_hbm.at[idx_v.at[j]],
                    raw_v.at[pl.ds(j * SLAB, SLAB)],
                    gsem,
                )
                for j in range(SLABS_PER_CHUNK)
            ]
            for cp in gathers:
                cp.wait()

            def transpose_slab(j, c):
                for bg in range(SLAB // LANES):
                    m = idx_v[j, pl.ds(bg * LANES, LANES)] != 0
                    rows = iot + (j * SLAB + bg * LANES)
                    pb0 = iot + (j * (8 * SLAB) + bg * LANES)
                    pb16 = pb0 + (2 * 8192)
                    for e0 in (0, LANES):
                        pb = pb0 if e0 == 0 else pb16
                        for k in range(LANES):
                            d_vec = (iot + k) & (LANES - 1)
                            pos_static = (((d_vec >> 3) << 13)
                                          + ((d_vec & 7) << 7))
                            vals = plsc.load_gather(raw_v, [rows, d_vec + e0])
                            plsc.store_scatter(
                                tout_v, [pos_static + pb],
                                jnp.where(m, vals, zeros16))
                return c

            lax.fori_loop(0, SLABS_PER_CHUNK, transpose_slab, 0)

            s_idx = chunk_slab // tb_dim
            qoff = (chunk_slab % tb_dim) * (8 * SLAB)
            writes = [
                pltpu.async_copy(
                    tout_v.at[pl.ds(te * (SLABS_PER_CHUNK * 8 * SLAB),
                                    SLABS_PER_CHUNK * 8 * SLAB)],
                    out_hbm.at[pl.ds(
                        s_idx * (te_dim * tb_dim * 8 * SLAB)
                        + te * (tb_dim * 8 * SLAB) + qoff,
                        SLABS_PER_CHUNK * 8 * SLAB)],
                    wsem,
                )
                for te in range(te_dim)
            ]
            for cp in writes:
                cp.wait()
            return carry

        lax.fori_loop(0, n_chunks, do_chunk, 0)

    return lookup


def kernel(instruction, table):
    b, s = instruction.shape
    vocab, embed = table.shape
    idx = instruction.astype(jnp.int32).T.reshape(s * (b // SLAB), SLAB)
    info = plsc.get_sparse_core_info()
    num_workers = info.num_cores * info.num_subcores
    flat = _build_lookup(s, b, embed, num_workers)(table, idx)
    out5 = flat.reshape(s, embed // 8, b // SLAB, 8, SLAB)
    return out5.transpose(2, 4, 0, 1, 3).reshape(b, s, embed)


# speedup vs baseline: 1.5001x; 1.0146x over previous
import functools

import jax
import jax.numpy as jnp
from jax import lax
from jax.experimental import pallas as pl
from jax.experimental.pallas import tpu as pltpu
from jax.experimental.pallas import tpu_sc as plsc

LANES = 16
SLAB = 128
SLABS_PER_CHUNK = 8
CHUNK = SLAB * SLABS_PER_CHUNK


def _build_lookup(s_dim: int, b_dim: int, embed: int, num_workers: int):
    tb_dim = b_dim // SLAB
    te_dim = embed // 8
    n_slabs = s_dim * tb_dim
    slabs_per_worker = n_slabs // num_workers
    n_chunks = slabs_per_worker // SLABS_PER_CHUNK
    n_pairs = (n_chunks - 1) // 2
    tout_len = te_dim * SLABS_PER_CHUNK * 8 * SLAB
    out_len = b_dim * s_dim * embed

    mesh = plsc.VectorSubcoreMesh(core_axis_name="c", subcore_axis_name="s")

    @functools.partial(
        pl.kernel,
        mesh=mesh,
        compiler_params=pltpu.CompilerParams(
            use_tc_tiling_on_sc=False, needs_layout_passes=False),
        out_type=jax.ShapeDtypeStruct((out_len,), jnp.float32),
        scratch_types=[
            pltpu.VMEM((2 * SLABS_PER_CHUNK, SLAB), jnp.int32),
            pltpu.VMEM((2 * CHUNK, embed), jnp.float32),
            pltpu.VMEM((tout_len,), jnp.float32),
            pltpu.SemaphoreType.DMA,
            pltpu.SemaphoreType.DMA,
            pltpu.SemaphoreType.DMA,
        ],
    )
    def lookup(table_hbm, idx_hbm, out_hbm, idx_v, raw_v, tout_v,
               gsem_a, gsem_b, wsem):
        n_cores = lax.axis_size("c")
        wid = lax.axis_index("s") * n_cores + lax.axis_index("c")
        slab_base = wid * slabs_per_worker
        iot = lax.iota(jnp.int32, LANES)
        zeros16 = jnp.zeros((LANES,), jnp.float32)

        def stage(chunk_slab, buf, gsem):
            pltpu.sync_copy(
                idx_hbm.at[pl.ds(chunk_slab, SLABS_PER_CHUNK)],
                idx_v.at[pl.ds(buf * SLABS_PER_CHUNK, SLABS_PER_CHUNK)])
            for j in range(SLABS_PER_CHUNK):
                pltpu.async_copy(
                    table_hbm.at[idx_v.at[buf * SLABS_PER_CHUNK + j]],
                    raw_v.at[pl.ds(buf * CHUNK + j * SLAB, SLAB)],
                    gsem,
                )

        def drain_gathers(gsem):
            for j in range(SLABS_PER_CHUNK):
                pltpu.make_async_copy(
                    table_hbm.at[idx_v.at[j]],
                    raw_v.at[pl.ds(j * SLAB, SLAB)],
                    gsem,
                ).wait()

        def process(chunk_slab, buf):
            def transpose_slab(j, c):
                for bg in range(SLAB // LANES):
                    m = idx_v[buf * SLABS_PER_CHUNK + j,
                              pl.ds(bg * LANES, LANES)] != 0
                    rows = iot + (buf * CHUNK + j * SLAB + bg * LANES)
                    pb0 = iot + (j * (8 * SLAB) + bg * LANES)
                    pb16 = pb0 + (2 * 8192)
                    for e0 in (0, LANES):
                        pb = pb0 if e0 == 0 else pb16
                        for k in range(LANES):
                            d_vec = (iot + k) & (LANES - 1)
                            pos_static = (((d_vec >> 3) << 13)
                                          + ((d_vec & 7) << 7))
                            vals = plsc.load_gather(raw_v, [rows, d_vec + e0])
                            plsc.store_scatter(
                                tout_v, [pos_static + pb],
                                jnp.where(m, vals, zeros16))
                return c

            lax.fori_loop(0, SLABS_PER_CHUNK, transpose_slab, 0)

            s_idx = chunk_slab // tb_dim
            qoff = (chunk_slab % tb_dim) * (8 * SLAB)
            writes = [
                pltpu.async_copy(
                    tout_v.at[pl.ds(te * (SLABS_PER_CHUNK * 8 * SLAB),
                                    SLABS_PER_CHUNK * 8 * SLAB)],
                    out_hbm.at[pl.ds(
                        s_idx * (te_dim * tb_dim * 8 * SLAB)
                        + te * (tb_dim * 8 * SLAB) + qoff,
                        SLABS_PER_CHUNK * 8 * SLAB)],
                    wsem,
                )
                for te in range(te_dim)
            ]
            for cp in writes:
                cp.wait()

        stage(slab_base, 0, gsem_a)

        def do_pair(t, carry):
            c0 = slab_base + (2 * t) * SLABS_PER_CHUNK
            c1 = c0 + SLABS_PER_CHUNK
            stage(c1, 1, gsem_b)
            drain_gathers(gsem_a)
            process(c0, 0)
            stage(c1 + SLABS_PER_CHUNK, 0, gsem_a)
            drain_gathers(gsem_b)
            process(c1, 1)
            return carry

        lax.fori_loop(0, n_pairs, do_pair, 0)
        drain_gathers(gsem_a)
        process(slab_base + (n_chunks - 1) * SLABS_PER_CHUNK, 0)

    return lookup


def kernel(instruction, table):
    b, s = instruction.shape
    vocab, embed = table.shape
    idx = instruction.astype(jnp.int32).T.reshape(s * (b // SLAB), SLAB)
    info = plsc.get_sparse_core_info()
    num_workers = info.num_cores * info.num_subcores
    flat = _build_lookup(s, b, embed, num_workers)(table, idx)
    out5 = flat.reshape(s, embed // 8, b // SLAB, 8, SLAB)
    return out5.transpose(2, 4, 0, 1, 3).reshape(b, s, embed)
